# baseline (speedup 1.0000x reference)
import jax
import jax.numpy as jnp
from jax.experimental import pallas as pl

N = 10000
D = 128
HID = 64
HEADS = 4


def _nearest_resize(x, size):
    d_in = x.shape[1]
    idx = (jnp.arange(size) * d_in) // size
    return x[:, idx]


def _gat_layer(h, src, dst, W, al, ar, bias, heads, out_feats, n):
    feat = (h @ W).reshape(n, heads, out_feats)
    el = jnp.sum(feat * al[None, :, :], axis=-1)
    er = jnp.sum(feat * ar[None, :, :], axis=-1)
    e = jax.nn.leaky_relu(el[src] + er[dst], negative_slope=0.2)
    m = jax.ops.segment_max(e, dst, num_segments=n)
    ee = jnp.exp(e - m[dst])
    s = jax.ops.segment_sum(ee, dst, num_segments=n)
    alpha = ee / s[dst]
    out = jax.ops.segment_sum(feat[src] * alpha[:, :, None], dst, num_segments=n)
    return out + bias.reshape(1, heads, out_feats)


def _heads_kernel(h2_ref, wa_ref, ba_ref, wm1_ref, bm1_ref, wm2_ref, bm2_ref,
                  wg1_ref, bg1_ref, wg2_ref, bg2_ref, temp_ref,
                  node_ref, graph_ref):
    h2 = h2_ref[...]
    temp = temp_ref[0, 0]
    a8 = h2 @ wa_ref[...] + ba_ref[...]
    attn = jax.nn.sigmoid(a8[:, 0:1])
    hg = jnp.sum(h2 * attn, axis=0, keepdims=True)
    n1 = jnp.maximum(h2 @ wm1_ref[...] + bm1_ref[...], 0.0)
    node_ref[...] = (n1 @ wm2_ref[...] + bm2_ref[...]) / temp
    g1 = jnp.maximum(hg @ wg1_ref[...] + bg1_ref[...], 0.0)
    g2 = (g1 @ wg2_ref[...] + bg2_ref[...]) / temp
    graph_ref[...] = jnp.broadcast_to(g2, (8, 8))


def kernel(rand_feat, func_emb, emb, edge_index, W_proj, b_proj, W1, al1, ar1,
           b1, W2, al2, ar2, b2, Wa, ba, Wm1, bm1, Wm2, bm2, Wg1, bg1, Wg2,
           bg2, temperature):
    n, d = emb.shape
    rf = _nearest_resize(rand_feat, d)
    fe = _nearest_resize(func_emb, d)
    h = jnp.concatenate([rf, fe, emb], axis=1)
    h = h @ W_proj + b_proj
    src = edge_index[0]
    dst = edge_index[1]
    h1 = _gat_layer(h, src, dst, W1, al1, ar1, b1, HEADS, HID, n).reshape(n, HEADS * HID)
    h2 = _gat_layer(h1, src, dst, W2, al2, ar2, b2, 1, HID, n).reshape(n, HID)

    pad = lambda w, k: jnp.pad(w, ((0, 0), (0, k - w.shape[1])))
    padv = lambda v, k: jnp.pad(v, (0, k - v.shape[0]))[None, :]
    node8, graph8 = pl.pallas_call(
        _heads_kernel,
        out_shape=(jax.ShapeDtypeStruct((N, 8), jnp.float32),
                   jax.ShapeDtypeStruct((8, 8), jnp.float32)),
    )(h2, pad(Wa, 8), padv(ba, 8), Wm1, bm1[None, :], pad(Wm2, 8), padv(bm2, 8),
      Wg1, bg1[None, :], pad(Wg2, 8), padv(bg2, 8),
      temperature.reshape(1, 1))
    return (node8[:, 0:2], graph8[0, 0:2])


# speedup vs baseline: 19.4802x; 19.4802x over previous
import dataclasses
import functools

import jax
import jax.numpy as jnp
from jax import lax
from jax.experimental import pallas as pl
from jax.experimental.pallas import tpu as pltpu
from jax.experimental.pallas import tpu_sc as plsc

N = 10000
E = 320000
D = 128
HID = 64
HEADS = 4

NC = 2
NS = 16
NW = NC * NS
EW = E // NW
K = 80
NCHUNK = EW // K
WB = 624
REM = N - NS * WB
ZR = 208
AUG = 80

_NEG = -1e30


def _tc0_body(x_ref, weff_ref, bp_ref, w1_ref, alr1_ref,
              faug_ref, elr_ref, m_ref):
    i = pl.program_id(0)
    x = x_ref[...]
    h = jnp.dot(x, weff_ref[...], preferred_element_type=jnp.float32)
    h = h + bp_ref[...]
    feat = jnp.dot(h, w1_ref[...], preferred_element_type=jnp.float32)
    elr = jnp.dot(feat, alr1_ref[...], preferred_element_type=jnp.float32, precision=jax.lax.Precision.HIGHEST)
    r = x.shape[0]
    ones = jnp.ones((r, 1), jnp.float32)
    zpad = jnp.zeros((r, AUG - HID - 2), jnp.float32)
    for hh in range(HEADS):
        slab = jnp.concatenate(
            [feat[:, hh * HID:(hh + 1) * HID], ones, elr[:, hh:hh + 1], zpad],
            axis=1)
        faug_ref[hh] = slab
    elr_ref[...] = elr
    bm = jnp.max(elr[:, 0:HEADS], axis=0)
    bmb = jnp.concatenate([bm, jnp.full((128 - HEADS,), _NEG, jnp.float32)])
    bmb = jnp.broadcast_to(bmb[None, :], (8, 128))
    m_ref[...] = jnp.where(i == 0, bmb, jnp.maximum(m_ref[...], bmb))


def _tc0(x, weff, bp, w1, alr1):
    grid = 5
    r = N // grid
    return pl.pallas_call(
        _tc0_body,
        grid=(grid,),
        in_specs=[
            pl.BlockSpec((r, 3 * D), lambda i: (i, 0)),
            pl.BlockSpec((3 * D, D), lambda i: (0, 0)),
            pl.BlockSpec((1, D), lambda i: (0, 0)),
            pl.BlockSpec((D, HEADS * HID), lambda i: (0, 0)),
            pl.BlockSpec((HEADS * HID, 8), lambda i: (0, 0)),
        ],
        out_specs=[
            pl.BlockSpec((HEADS, r, AUG), lambda i: (0, i, 0)),
            pl.BlockSpec((r, 8), lambda i: (i, 0)),
            pl.BlockSpec((8, 128), lambda i: (0, 0)),
        ],
        out_shape=[
            jax.ShapeDtypeStruct((HEADS, N, AUG), jnp.float32),
            jax.ShapeDtypeStruct((N, 8), jnp.float32),
            jax.ShapeDtypeStruct((8, 128), jnp.float32),
        ],
    )(x, weff, bp, w1, alr1)


def _tc1_body(acc_ref, b1_ref, w2_ref, alr2_ref, faug_ref, elr_ref, m_ref):
    i = pl.program_id(0)
    parts = []
    for hh in range(HEADS):
        num = acc_ref[0, hh, :, 0:HID] + acc_ref[1, hh, :, 0:HID]
        s = acc_ref[0, hh, :, HID:HID + 1] + acc_ref[1, hh, :, HID:HID + 1]
        safe = jnp.where(s > 0.0, s, 1.0)
        parts.append(jnp.where(s > 0.0, num / safe, 0.0)
                     + b1_ref[:, hh * HID:(hh + 1) * HID])
    h1 = jnp.concatenate(parts, axis=1)
    feat2 = jnp.dot(h1, w2_ref[...], preferred_element_type=jnp.float32)
    elr2 = jnp.dot(feat2, alr2_ref[...], preferred_element_type=jnp.float32, precision=jax.lax.Precision.HIGHEST)
    r = h1.shape[0]
    ones = jnp.ones((r, 1), jnp.float32)
    zpad = jnp.zeros((r, AUG - HID - 2), jnp.float32)
    faug_ref[...] = jnp.concatenate([feat2, ones, elr2[:, 0:1], zpad], axis=1)
    elr_ref[...] = elr2
    bm = jnp.max(elr2[:, 0:1], axis=0)
    bmb = jnp.concatenate([bm, jnp.full((127,), _NEG, jnp.float32)])
    bmb = jnp.broadcast_to(bmb[None, :], (8, 128))
    m_ref[...] = jnp.where(i == 0, bmb, jnp.maximum(m_ref[...], bmb))


def _tc1(acc, b1, w2, alr2):
    grid = 5
    r = N // grid
    return pl.pallas_call(
        _tc1_body,
        grid=(grid,),
        in_specs=[
            pl.BlockSpec((NC, HEADS, r, AUG), lambda i: (0, 0, i, 0)),
            pl.BlockSpec((1, HEADS * HID), lambda i: (0, 0)),
            pl.BlockSpec((HEADS * HID, HID), lambda i: (0, 0)),
            pl.BlockSpec((HID, 8), lambda i: (0, 0)),
        ],
        out_specs=[
            pl.BlockSpec((r, AUG), lambda i: (i, 0)),
            pl.BlockSpec((r, 8), lambda i: (i, 0)),
            pl.BlockSpec((8, 128), lambda i: (0, 0)),
        ],
        out_shape=[
            jax.ShapeDtypeStruct((N, AUG), jnp.float32),
            jax.ShapeDtypeStruct((N, 8), jnp.float32),
            jax.ShapeDtypeStruct((8, 128), jnp.float32),
        ],
    )(acc, b1, w2, alr2)


def _tc2_body(acc_ref, b2_ref, wa_ref, ba_ref, wm1_ref, bm1_ref, wm2_ref,
              bm2_ref, wg1_ref, bg1_ref, wg2_ref, bg2_ref, temp_ref,
              node_ref, graph_ref):
    num = acc_ref[0, :, 0:HID] + acc_ref[1, :, 0:HID]
    s = acc_ref[0, :, HID:HID + 1] + acc_ref[1, :, HID:HID + 1]
    safe = jnp.where(s > 0.0, s, 1.0)
    h2 = jnp.where(s > 0.0, num / safe, 0.0) + b2_ref[...]
    temp = temp_ref[0, 0]
    a8 = jnp.dot(h2, wa_ref[...], preferred_element_type=jnp.float32)
    attn = jax.nn.sigmoid(a8[:, 0:1] + ba_ref[0, 0])
    hg = jnp.sum(h2 * attn, axis=0, keepdims=True)
    n1 = jnp.maximum(
        jnp.dot(h2, wm1_ref[...], preferred_element_type=jnp.float32)
        + bm1_ref[...], 0.0)
    node_ref[...] = (jnp.dot(n1, wm2_ref[...],
                             preferred_element_type=jnp.float32)
                     + bm2_ref[...]) / temp
    g1 = jnp.maximum(
        jnp.dot(hg, wg1_ref[...], preferred_element_type=jnp.float32)
        + bg1_ref[...], 0.0)
    g2 = (jnp.dot(g1, wg2_ref[...], preferred_element_type=jnp.float32)
          + bg2_ref[...]) / temp
    graph_ref[...] = jnp.broadcast_to(g2, (8, 8))


def _tc2(acc, b2, wa, ba, wm1, bm1, wm2, bm2, wg1, bg1, wg2, bg2, temp):
    return pl.pallas_call(
        _tc2_body,
        out_shape=[
            jax.ShapeDtypeStruct((N, 8), jnp.float32),
            jax.ShapeDtypeStruct((8, 8), jnp.float32),
        ],
    )(acc, b2, wa, ba, wm1, bm1, wm2, bm2, wg1, bg1, wg2, bg2, temp)


def _edge_phase(faug, elr, src, dst, heads, er_col):
    mesh = plsc.VectorSubcoreMesh(core_axis_name="c", subcore_axis_name="s")
    cp = pltpu.CompilerParams(needs_layout_passes=False,
                              use_tc_tiling_on_sc=False)

    @functools.partial(
        pl.kernel,
        mesh=mesh,
        compiler_params=cp,
        out_type=jax.ShapeDtypeStruct((NC, heads, N, AUG), jnp.float32),
        scratch_types=[
            pltpu.VMEM((K,), jnp.int32),
            pltpu.VMEM((K,), jnp.int32),
            pltpu.VMEM((K,), jnp.float32),
            pltpu.VMEM((K, AUG), jnp.float32),
            pltpu.VMEM((K, 16), jnp.float32),
            pltpu.VMEM((ZR, AUG), jnp.float32),
            pltpu.VMEM_SHARED((N, AUG), jnp.float32),
        ],
    )
    def sc_kernel(faug_hbm, elr_hbm, src_hbm, dst_hbm, out_hbm,
                  srcb, dstb, eeb, rows, err, zbuf, accum):
        cid = lax.axis_index("c")
        sid = lax.axis_index("s")
        wid = cid * NS + sid
        base0 = wid * EW

        @pl.loop(0, ZR)
        def _zero(r):
            for c in range(AUG // 16):
                zbuf[r, pl.ds(c * 16, 16)] = jnp.zeros((16,), jnp.float32)

        iota16 = lax.iota(jnp.int32, 16)

        for hh in range(heads):
            for z in range(WB // ZR):
                pltpu.sync_copy(
                    zbuf, accum.at[pl.ds(sid * WB + z * ZR, ZR)])

            @pl.when(sid == NS - 1)
            def _zrem():
                pltpu.sync_copy(zbuf.at[pl.ds(0, REM)],
                                accum.at[pl.ds(NS * WB, REM)])

            plsc.subcore_barrier()

            ecol = jnp.full((16,), er_col + hh, jnp.int32)
            mcol = jnp.full((16,), 8 + hh, jnp.int32)

            @pl.loop(0, NCHUNK)
            def _chunk(ci):
                base = base0 + ci * K
                pltpu.sync_copy(src_hbm.at[pl.ds(base, K)], srcb)
                pltpu.sync_copy(dst_hbm.at[pl.ds(base, K)], dstb)
                if hh > 0:
                    for c in range(K // 16):
                        sl = pl.ds(c * 16, 16)
                        srcb[sl] = srcb[sl] + jnp.full((16,), hh * N,
                                                       jnp.int32)
                pltpu.sync_copy(faug_hbm.at[srcb], rows)
                pltpu.sync_copy(elr_hbm.at[dstb], err)
                for c in range(K // 16):
                    sl = pl.ds(c * 16, 16)
                    rowv = iota16 + c * 16
                    erv = plsc.load_gather(err, [rowv, ecol])
                    mhv = plsc.load_gather(err, [rowv, mcol])
                    elv = plsc.load_gather(
                        rows, [rowv, jnp.full((16,), HID + 1, jnp.int32)])
                    x = elv + erv
                    e = jnp.maximum(x, x * 0.2)
                    ca = erv + mhv
                    cc = jnp.maximum(ca, ca * 0.2)
                    eeb[sl] = jnp.exp(e - cc)

                @pl.loop(0, K)
                def _scale(j):
                    eev = plsc.load_gather(eeb, [jnp.full((16,), j,
                                                          jnp.int32)])
                    for c in range(AUG // 16):
                        sl = pl.ds(c * 16, 16)
                        rows[j, sl] = rows[j, sl] * eev

                pltpu.sync_copy(rows, accum.at[dstb], add=True)

            plsc.subcore_barrier()
            pltpu.sync_copy(accum.at[pl.ds(sid * WB, WB)],
                            out_hbm.at[cid, hh, pl.ds(sid * WB, WB)])

            @pl.when(sid == NS - 1)
            def _wrem():
                pltpu.sync_copy(accum.at[pl.ds(NS * WB, REM)],
                                out_hbm.at[cid, hh, pl.ds(NS * WB, REM)])

    return sc_kernel(faug, elr, src, dst)


def kernel(rand_feat, func_emb, emb, edge_index, W_proj, b_proj, W1, al1, ar1,
           b1, W2, al2, ar2, b2, Wa, ba, Wm1, bm1, Wm2, bm2, Wg1, bg1, Wg2,
           bg2, temperature):
    f32 = jnp.float32
    idx_rf = (jnp.arange(D) * rand_feat.shape[1]) // D
    idx_fe = (jnp.arange(D) * func_emb.shape[1]) // D
    x = jnp.concatenate([rand_feat[:, idx_rf], func_emb[:, idx_fe], emb],
                        axis=1)

    alr1 = jnp.zeros((HEADS * HID, 8), f32)
    for hh in range(HEADS):
        alr1 = alr1.at[hh * HID:(hh + 1) * HID, hh].set(al1[hh])
        alr1 = alr1.at[hh * HID:(hh + 1) * HID, 4 + hh].set(ar1[hh])
    alr2 = jnp.zeros((HID, 8), f32)
    alr2 = alr2.at[:, 0].set(al2[0])
    alr2 = alr2.at[:, 1].set(ar2[0])

    pad8 = lambda w: jnp.pad(w, ((0, 0), (0, 8 - w.shape[1])))
    padv8 = lambda v: jnp.pad(v, (0, 8 - v.shape[0]))[None, :]

    src = edge_index[0]
    dst = edge_index[1]

    faug1, elr1, m1 = _tc0(x, W_proj, b_proj[None, :], W1, alr1)
    elr1_16 = jnp.concatenate(
        [elr1, jnp.broadcast_to(m1[0:1, 0:HEADS], (N, HEADS)),
         jnp.zeros((N, 4), jnp.float32)], axis=1)

    acc1 = _edge_phase(faug1.reshape(HEADS * N, AUG), elr1_16,
                       src, dst, HEADS, er_col=4)

    faug2, elr2, m2 = _tc1(acc1, b1[None, :], W2, alr2)
    elr2_16 = jnp.concatenate(
        [elr2, jnp.broadcast_to(m2[0:1, 0:1], (N, 1)),
         jnp.zeros((N, 7), jnp.float32)], axis=1)

    acc2 = _edge_phase(faug2, elr2_16, src, dst, 1, er_col=1)

    node8, graph8 = _tc2(
        acc2.reshape(NC, N, AUG), b2[None, :], pad8(Wa), padv8(ba),
        Wm1, bm1[None, :], pad8(Wm2), padv8(bm2),
        Wg1, bg1[None, :], pad8(Wg2), padv8(bg2),
        temperature.reshape(1, 1).astype(f32))
    return (node8[:, 0:2], graph8[0, 0:2])


# speedup vs baseline: 45.4738x; 2.3344x over previous
import dataclasses
import functools

import jax
import jax.numpy as jnp
from jax import lax
from jax.experimental import pallas as pl
from jax.experimental.pallas import tpu as pltpu
from jax.experimental.pallas import tpu_sc as plsc

N = 10000
E = 320000
D = 128
HID = 64
HEADS = 4

NC = 2
NS = 16
NW = NC * NS
EW = E // NW
K = 80
NCHUNK = EW // K
WB = 624
REM = N - NS * WB
ZR = 208
NBUF = 5
AUG = 80

_NEG = -1e30


def _tc0_body(x_ref, weff_ref, bp_ref, w1_ref, alr1_ref,
              faug_ref, elr_ref, m_ref):
    i = pl.program_id(0)
    x = x_ref[...]
    h = jnp.dot(x, weff_ref[...], preferred_element_type=jnp.float32)
    h = h + bp_ref[...]
    feat = jnp.dot(h, w1_ref[...], preferred_element_type=jnp.float32)
    elr = jnp.dot(feat, alr1_ref[...], preferred_element_type=jnp.float32, precision=jax.lax.Precision.HIGHEST)
    r = x.shape[0]
    ones = jnp.ones((r, 1), jnp.float32)
    zpad = jnp.zeros((r, AUG - HID - 2), jnp.float32)
    for hh in range(HEADS):
        slab = jnp.concatenate(
            [feat[:, hh * HID:(hh + 1) * HID], ones, elr[:, hh:hh + 1], zpad],
            axis=1)
        faug_ref[hh] = slab
    elr_ref[...] = elr
    bm = jnp.max(elr[:, 0:HEADS], axis=0)
    bmb = jnp.concatenate([bm, jnp.full((128 - HEADS,), _NEG, jnp.float32)])
    bmb = jnp.broadcast_to(bmb[None, :], (8, 128))
    m_ref[...] = jnp.where(i == 0, bmb, jnp.maximum(m_ref[...], bmb))


def _tc0(x, weff, bp, w1, alr1):
    grid = 5
    r = N // grid
    return pl.pallas_call(
        _tc0_body,
        grid=(grid,),
        in_specs=[
            pl.BlockSpec((r, 3 * D), lambda i: (i, 0)),
            pl.BlockSpec((3 * D, D), lambda i: (0, 0)),
            pl.BlockSpec((1, D), lambda i: (0, 0)),
            pl.BlockSpec((D, HEADS * HID), lambda i: (0, 0)),
            pl.BlockSpec((HEADS * HID, 8), lambda i: (0, 0)),
        ],
        out_specs=[
            pl.BlockSpec((HEADS, r, AUG), lambda i: (0, i, 0)),
            pl.BlockSpec((r, 8), lambda i: (i, 0)),
            pl.BlockSpec((8, 128), lambda i: (0, 0)),
        ],
        out_shape=[
            jax.ShapeDtypeStruct((HEADS, N, AUG), jnp.float32),
            jax.ShapeDtypeStruct((N, 8), jnp.float32),
            jax.ShapeDtypeStruct((8, 128), jnp.float32),
        ],
    )(x, weff, bp, w1, alr1)


def _tc1_body(acc_ref, b1_ref, w2_ref, alr2_ref, faug_ref, elr_ref, m_ref):
    i = pl.program_id(0)
    parts = []
    for hh in range(HEADS):
        num = acc_ref[0, hh, :, 0:HID] + acc_ref[1, hh, :, 0:HID]
        s = acc_ref[0, hh, :, HID:HID + 1] + acc_ref[1, hh, :, HID:HID + 1]
        safe = jnp.where(s > 0.0, s, 1.0)
        parts.append(jnp.where(s > 0.0, num / safe, 0.0)
                     + b1_ref[:, hh * HID:(hh + 1) * HID])
    h1 = jnp.concatenate(parts, axis=1)
    feat2 = jnp.dot(h1, w2_ref[...], preferred_element_type=jnp.float32)
    elr2 = jnp.dot(feat2, alr2_ref[...], preferred_element_type=jnp.float32, precision=jax.lax.Precision.HIGHEST)
    r = h1.shape[0]
    ones = jnp.ones((r, 1), jnp.float32)
    zpad = jnp.zeros((r, AUG - HID - 2), jnp.float32)
    faug_ref[...] = jnp.concatenate([feat2, ones, elr2[:, 0:1], zpad], axis=1)
    elr_ref[...] = elr2
    bm = jnp.max(elr2[:, 0:1], axis=0)
    bmb = jnp.concatenate([bm, jnp.full((127,), _NEG, jnp.float32)])
    bmb = jnp.broadcast_to(bmb[None, :], (8, 128))
    m_ref[...] = jnp.where(i == 0, bmb, jnp.maximum(m_ref[...], bmb))


def _tc1(acc, b1, w2, alr2):
    grid = 5
    r = N // grid
    return pl.pallas_call(
        _tc1_body,
        grid=(grid,),
        in_specs=[
            pl.BlockSpec((NC, HEADS, r, AUG), lambda i: (0, 0, i, 0)),
            pl.BlockSpec((1, HEADS * HID), lambda i: (0, 0)),
            pl.BlockSpec((HEADS * HID, HID), lambda i: (0, 0)),
            pl.BlockSpec((HID, 8), lambda i: (0, 0)),
        ],
        out_specs=[
            pl.BlockSpec((r, AUG), lambda i: (i, 0)),
            pl.BlockSpec((r, 8), lambda i: (i, 0)),
            pl.BlockSpec((8, 128), lambda i: (0, 0)),
        ],
        out_shape=[
            jax.ShapeDtypeStruct((N, AUG), jnp.float32),
            jax.ShapeDtypeStruct((N, 8), jnp.float32),
            jax.ShapeDtypeStruct((8, 128), jnp.float32),
        ],
    )(acc, b1, w2, alr2)


def _tc2_body(acc_ref, b2_ref, wa_ref, ba_ref, wm1_ref, bm1_ref, wm2_ref,
              bm2_ref, wg1_ref, bg1_ref, wg2_ref, bg2_ref, temp_ref,
              node_ref, graph_ref):
    num = acc_ref[0, :, 0:HID] + acc_ref[1, :, 0:HID]
    s = acc_ref[0, :, HID:HID + 1] + acc_ref[1, :, HID:HID + 1]
    safe = jnp.where(s > 0.0, s, 1.0)
    h2 = jnp.where(s > 0.0, num / safe, 0.0) + b2_ref[...]
    temp = temp_ref[0, 0]
    a8 = jnp.dot(h2, wa_ref[...], preferred_element_type=jnp.float32)
    attn = jax.nn.sigmoid(a8[:, 0:1] + ba_ref[0, 0])
    hg = jnp.sum(h2 * attn, axis=0, keepdims=True)
    n1 = jnp.maximum(
        jnp.dot(h2, wm1_ref[...], preferred_element_type=jnp.float32)
        + bm1_ref[...], 0.0)
    node_ref[...] = (jnp.dot(n1, wm2_ref[...],
                             preferred_element_type=jnp.float32)
                     + bm2_ref[...]) / temp
    g1 = jnp.maximum(
        jnp.dot(hg, wg1_ref[...], preferred_element_type=jnp.float32)
        + bg1_ref[...], 0.0)
    g2 = (jnp.dot(g1, wg2_ref[...], preferred_element_type=jnp.float32)
          + bg2_ref[...]) / temp
    graph_ref[...] = jnp.broadcast_to(g2, (8, 8))


def _tc2(acc, b2, wa, ba, wm1, bm1, wm2, bm2, wg1, bg1, wg2, bg2, temp):
    return pl.pallas_call(
        _tc2_body,
        out_shape=[
            jax.ShapeDtypeStruct((N, 8), jnp.float32),
            jax.ShapeDtypeStruct((8, 8), jnp.float32),
        ],
    )(acc, b2, wa, ba, wm1, bm1, wm2, bm2, wg1, bg1, wg2, bg2, temp)


def _edge_phase(faug, elr, src, dst, heads, er_col):
    mesh = plsc.VectorSubcoreMesh(core_axis_name="c", subcore_axis_name="s")
    cp = pltpu.CompilerParams(needs_layout_passes=False,
                              use_tc_tiling_on_sc=False)

    buf_types = []
    for _ in range(NBUF):
        buf_types += [
            pltpu.VMEM((K,), jnp.int32),
            pltpu.VMEM((K,), jnp.int32),
            pltpu.VMEM((K, AUG), jnp.float32),
            pltpu.VMEM((K, 16), jnp.float32),
        ]
    sem_types = [pltpu.SemaphoreType.DMA] * (2 * NBUF)

    @functools.partial(
        pl.kernel,
        mesh=mesh,
        compiler_params=cp,
        out_type=jax.ShapeDtypeStruct((NC, heads, N, AUG), jnp.float32),
        scratch_types=buf_types + [
            pltpu.VMEM((K,), jnp.float32),
            pltpu.VMEM((ZR, AUG), jnp.float32),
            pltpu.VMEM_SHARED((N, AUG), jnp.float32),
        ] + sem_types,
    )
    def sc_kernel(faug_hbm, elr_hbm, src_hbm, dst_hbm, out_hbm, *scr):
        srcbs = [scr[4 * b + 0] for b in range(NBUF)]
        dstbs = [scr[4 * b + 1] for b in range(NBUF)]
        rowss = [scr[4 * b + 2] for b in range(NBUF)]
        errs = [scr[4 * b + 3] for b in range(NBUF)]
        eeb, zbuf, accum = scr[4 * NBUF:4 * NBUF + 3]
        lsem = list(scr[4 * NBUF + 3:4 * NBUF + 3 + NBUF])
        ssem = list(scr[4 * NBUF + 3 + NBUF:])

        cid = lax.axis_index("c")
        sid = lax.axis_index("s")
        wid = cid * NS + sid
        base0 = wid * EW

        @pl.loop(0, ZR)
        def _zero(r):
            for c in range(AUG // 16):
                zbuf[r, pl.ds(c * 16, 16)] = jnp.zeros((16,), jnp.float32)

        iota16 = lax.iota(jnp.int32, 16)

        for hh in range(heads):
            for z in range(WB // ZR):
                pltpu.sync_copy(
                    zbuf, accum.at[pl.ds(sid * WB + z * ZR, ZR)])

            @pl.when(sid == NS - 1)
            def _zrem():
                pltpu.sync_copy(zbuf.at[pl.ds(0, REM)],
                                accum.at[pl.ds(NS * WB, REM)])

            plsc.subcore_barrier()

            ecol = jnp.full((16,), er_col + hh, jnp.int32)
            mcol = jnp.full((16,), 8 + hh, jnp.int32)

            @pl.loop(0, NCHUNK // NBUF)
            def _round(i):
                base_r = base0 + i * (NBUF * K)
                icps = []
                for b in range(NBUF):
                    @pl.when(i > 0)
                    def _wprev(b=b):
                        pltpu.make_async_copy(
                            rowss[b], accum.at[dstbs[b]], ssem[b]).wait()
                    icps.append((
                        pltpu.async_copy(
                            src_hbm.at[pl.ds(base_r + b * K, K)],
                            srcbs[b], lsem[b]),
                        pltpu.async_copy(
                            dst_hbm.at[pl.ds(base_r + b * K, K)],
                            dstbs[b], lsem[b])))
                gcps = []
                for b in range(NBUF):
                    ia, ib = icps[b]
                    ia.wait()
                    ib.wait()
                    if hh > 0:
                        for c in range(K // 16):
                            sl = pl.ds(c * 16, 16)
                            srcbs[b][sl] = srcbs[b][sl] + jnp.full(
                                (16,), hh * N, jnp.int32)
                    gcps.append((
                        pltpu.async_copy(faug_hbm.at[srcbs[b]], rowss[b],
                                         lsem[b]),
                        pltpu.async_copy(elr_hbm.at[dstbs[b]], errs[b],
                                         lsem[b])))
                for b in range(NBUF):
                    ga, ge = gcps[b]
                    ga.wait()
                    ge.wait()
                    rows = rowss[b]
                    err = errs[b]
                    for c in range(K // 16):
                        sl = pl.ds(c * 16, 16)
                        rowv = iota16 + c * 16
                        erv = plsc.load_gather(err, [rowv, ecol])
                        mhv = plsc.load_gather(err, [rowv, mcol])
                        elv = plsc.load_gather(
                            rows, [rowv, jnp.full((16,), HID + 1,
                                                  jnp.int32)])
                        x = elv + erv
                        e = jnp.maximum(x, x * 0.2)
                        ca = erv + mhv
                        cc = jnp.maximum(ca, ca * 0.2)
                        eeb[sl] = jnp.exp(e - cc)

                    @pl.loop(0, K)
                    def _scale(j):
                        eev = plsc.load_gather(eeb, [jnp.full((16,), j,
                                                              jnp.int32)])
                        for c in range(AUG // 16):
                            sl = pl.ds(c * 16, 16)
                            rows[j, sl] = rows[j, sl] * eev

                    pltpu.async_copy(rows, accum.at[dstbs[b]], ssem[b],
                                     add=True)

            for b in range(NBUF):
                pltpu.make_async_copy(
                    rowss[b], accum.at[dstbs[b]], ssem[b]).wait()

            plsc.subcore_barrier()
            pltpu.sync_copy(accum.at[pl.ds(sid * WB, WB)],
                            out_hbm.at[cid, hh, pl.ds(sid * WB, WB)])

            @pl.when(sid == NS - 1)
            def _wrem():
                pltpu.sync_copy(accum.at[pl.ds(NS * WB, REM)],
                                out_hbm.at[cid, hh, pl.ds(NS * WB, REM)])

    return sc_kernel(faug, elr, src, dst)


def kernel(rand_feat, func_emb, emb, edge_index, W_proj, b_proj, W1, al1, ar1,
           b1, W2, al2, ar2, b2, Wa, ba, Wm1, bm1, Wm2, bm2, Wg1, bg1, Wg2,
           bg2, temperature):
    f32 = jnp.float32
    idx_rf = (jnp.arange(D) * rand_feat.shape[1]) // D
    idx_fe = (jnp.arange(D) * func_emb.shape[1]) // D
    x = jnp.concatenate([rand_feat[:, idx_rf], func_emb[:, idx_fe], emb],
                        axis=1)

    alr1 = jnp.zeros((HEADS * HID, 8), f32)
    for hh in range(HEADS):
        alr1 = alr1.at[hh * HID:(hh + 1) * HID, hh].set(al1[hh])
        alr1 = alr1.at[hh * HID:(hh + 1) * HID, 4 + hh].set(ar1[hh])
    alr2 = jnp.zeros((HID, 8), f32)
    alr2 = alr2.at[:, 0].set(al2[0])
    alr2 = alr2.at[:, 1].set(ar2[0])

    pad8 = lambda w: jnp.pad(w, ((0, 0), (0, 8 - w.shape[1])))
    padv8 = lambda v: jnp.pad(v, (0, 8 - v.shape[0]))[None, :]

    src = edge_index[0]
    dst = edge_index[1]

    faug1, elr1, m1 = _tc0(x, W_proj, b_proj[None, :], W1, alr1)
    elr1_16 = jnp.concatenate(
        [elr1, jnp.broadcast_to(m1[0:1, 0:HEADS], (N, HEADS)),
         jnp.zeros((N, 4), jnp.float32)], axis=1)

    acc1 = _edge_phase(faug1.reshape(HEADS * N, AUG), elr1_16,
                       src, dst, HEADS, er_col=4)

    faug2, elr2, m2 = _tc1(acc1, b1[None, :], W2, alr2)
    elr2_16 = jnp.concatenate(
        [elr2, jnp.broadcast_to(m2[0:1, 0:1], (N, 1)),
         jnp.zeros((N, 7), jnp.float32)], axis=1)

    acc2 = _edge_phase(faug2, elr2_16, src, dst, 1, er_col=1)

    node8, graph8 = _tc2(
        acc2.reshape(NC, N, AUG), b2[None, :], pad8(Wa), padv8(ba),
        Wm1, bm1[None, :], pad8(Wm2), padv8(bm2),
        Wg1, bg1[None, :], pad8(Wg2), padv8(bg2),
        temperature.reshape(1, 1).astype(f32))
    return (node8[:, 0:2], graph8[0, 0:2])
